# Initial kernel scaffold; baseline (speedup 1.0000x reference)
import functools

import jax
import jax.numpy as jnp
from jax import lax
from jax.experimental import pallas as pl
from jax.experimental.pallas import tpu as pltpu
from jax.experimental.pallas import tpu_sc as plsc

DIM = 32
NC = 2
NS = 16
NW = NC * NS
SPW = 128
K = 8
CH = K * SPW


@functools.lru_cache(maxsize=None)
def _make_gather(batch_flat: int, vocab: int):
    assert batch_flat % (NW * CH) == 0, batch_flat
    per_w = batch_flat // NW
    nchunk = per_w // CH
    mesh = plsc.VectorSubcoreMesh(
        core_axis_name="c", subcore_axis_name="s",
        num_cores=NC, num_subcores=NS,
    )

    @functools.partial(
        pl.kernel,
        out_type=jax.ShapeDtypeStruct((batch_flat, DIM), jnp.float32),
        mesh=mesh,
        scratch_types=[
            pltpu.VMEM((per_w,), jnp.int32),
            pltpu.VMEM((CH, DIM), jnp.float32),
            pltpu.SemaphoreType.DMA,
        ],
    )
    def k(idx_hbm, table_hbm, out_hbm, idx_v, rows_v, gsem):
        wid = lax.axis_index("s") * NC + lax.axis_index("c")
        base = wid * per_w
        pltpu.sync_copy(idx_hbm.at[pl.ds(base, per_w)], idx_v)

        @pl.loop(0, nchunk)
        def _chunk(c):
            off = c * CH
            cps = [
                pltpu.async_copy(
                    table_hbm.at[idx_v.at[pl.ds(off + j * SPW, SPW)]],
                    rows_v.at[pl.ds(j * SPW, SPW)],
                    gsem,
                )
                for j in range(K)
            ]
            for cp in cps:
                cp.wait()
            pltpu.sync_copy(rows_v, out_hbm.at[pl.ds(base + off, CH)])

    return k


def kernel(input, table):
    b, w = input.shape
    vocab, dim = table.shape
    assert dim == DIM
    idx = input.reshape(b * w).astype(jnp.int32)
    out = _make_gather(b * w, vocab)(idx, table)
    return out.reshape(b, w, dim)


# speedup vs baseline: 1.5601x; 1.5601x over previous
import functools

import jax
import jax.numpy as jnp
from jax import lax
from jax.experimental import pallas as pl
from jax.experimental.pallas import tpu as pltpu
from jax.experimental.pallas import tpu_sc as plsc

DIM = 32
NC = 2
NS = 16
NW = NC * NS
SPW = 128
K = 8
CH = K * SPW


@functools.lru_cache(maxsize=None)
def _make_gather(batch_flat: int, vocab: int):
    assert batch_flat % (NW * CH) == 0, batch_flat
    per_w = batch_flat // NW
    nchunk = per_w // CH
    mesh = plsc.VectorSubcoreMesh(
        core_axis_name="c", subcore_axis_name="s",
        num_cores=NC, num_subcores=NS,
    )

    @functools.partial(
        pl.kernel,
        out_type=jax.ShapeDtypeStruct((batch_flat, DIM), jnp.float32),
        mesh=mesh,
        scratch_types=[
            pltpu.VMEM((per_w,), jnp.int32),
            pltpu.VMEM((CH, DIM), jnp.float32),
            pltpu.SemaphoreType.DMA,
        ],
        compiler_params=pltpu.CompilerParams(use_tc_tiling_on_sc=False),
    )
    def k(idx_hbm, table_hbm, out_hbm, idx_v, rows_v, gsem):
        wid = lax.axis_index("s") * NC + lax.axis_index("c")
        base = wid * per_w
        pltpu.sync_copy(idx_hbm.at[pl.ds(base, per_w)], idx_v)

        @pl.loop(0, nchunk)
        def _chunk(c):
            off = c * CH
            cps = [
                pltpu.async_copy(
                    table_hbm.at[idx_v.at[pl.ds(off + j * SPW, SPW)]],
                    rows_v.at[pl.ds(j * SPW, SPW)],
                    gsem,
                )
                for j in range(K)
            ]
            for cp in cps:
                cp.wait()
            pltpu.sync_copy(rows_v, out_hbm.at[pl.ds(base + off, CH)])

    return k


def kernel(input, table):
    b, w = input.shape
    vocab, dim = table.shape
    assert dim == DIM
    idx = input.reshape(b * w).astype(jnp.int32)
    out = _make_gather(b * w, vocab)(idx, table)
    return out.reshape(b, w, dim)


# speedup vs baseline: 1.5731x; 1.0083x over previous
import functools

import jax
import jax.numpy as jnp
from jax import lax
from jax.experimental import pallas as pl
from jax.experimental.pallas import tpu as pltpu
from jax.experimental.pallas import tpu_sc as plsc

DIM = 32
NC = 2
NS = 16
NW = NC * NS
SPW = 128
K = 4
CH = K * SPW


@functools.lru_cache(maxsize=None)
def _make_gather(batch_flat: int, vocab: int):
    assert batch_flat % (NW * CH) == 0, batch_flat
    per_w = batch_flat // NW
    nchunk = per_w // CH
    assert nchunk % 2 == 0, nchunk
    mesh = plsc.VectorSubcoreMesh(
        core_axis_name="c", subcore_axis_name="s",
        num_cores=NC, num_subcores=NS,
    )

    @functools.partial(
        pl.kernel,
        out_type=jax.ShapeDtypeStruct((batch_flat, DIM), jnp.float32),
        mesh=mesh,
        scratch_types=[
            pltpu.VMEM((per_w,), jnp.int32),
            pltpu.VMEM((2, CH, DIM), jnp.float32),
            pltpu.SemaphoreType.DMA,
            pltpu.SemaphoreType.DMA,
        ],
        compiler_params=pltpu.CompilerParams(use_tc_tiling_on_sc=False),
    )
    def k(idx_hbm, table_hbm, out_hbm, idx_v, rows_v, sem0, sem1):
        sems = (sem0, sem1)
        wid = lax.axis_index("s") * NC + lax.axis_index("c")
        base = wid * per_w
        pltpu.sync_copy(idx_hbm.at[pl.ds(base, per_w)], idx_v)

        def fire(c, slot):
            off = c * CH
            for j in range(K):
                pltpu.async_copy(
                    table_hbm.at[idx_v.at[pl.ds(off + j * SPW, SPW)]],
                    rows_v.at[slot, pl.ds(j * SPW, SPW)],
                    sems[slot],
                )

        def drain(c, slot):
            off = c * CH
            for j in range(K):
                pltpu.make_async_copy(
                    table_hbm.at[idx_v.at[pl.ds(off + j * SPW, SPW)]],
                    rows_v.at[slot, pl.ds(j * SPW, SPW)],
                    sems[slot],
                ).wait()

        fire(0, 0)

        @pl.loop(0, nchunk, step=2)
        def _pair(c0):
            for b in range(2):
                c = c0 + b

                @pl.when(c + 1 < nchunk)
                def _():
                    fire(c + 1, 1 - b)

                drain(c, b)
                pltpu.sync_copy(rows_v.at[b], out_hbm.at[pl.ds(base + c * CH, CH)])

    return k


def kernel(input, table):
    b, w = input.shape
    vocab, dim = table.shape
    assert dim == DIM
    idx = input.reshape(b * w).astype(jnp.int32)
    out = _make_gather(b * w, vocab)(idx, table)
    return out.reshape(b, w, dim)
